# Initial kernel scaffold; baseline (speedup 1.0000x reference)
import functools

import jax
import jax.numpy as jnp
from jax import lax
from jax.experimental import pallas as pl
from jax.experimental.pallas import tpu as pltpu
from jax.experimental.pallas import tpu_sc as plsc

_VOCAB = 1000000
_D = 64
_B = 16384
_SEQ = 200

_NC = 2
_NS = 16
_NW = _NC * _NS
_BPW = _B // _NW
_CHUNK = 128
_NCHUNK = _BPW // _CHUNK
_JC = 50
_NJ = _SEQ // _JC


def _sc_pool(text_r, emb):
  mesh = plsc.VectorSubcoreMesh(core_axis_name="c", subcore_axis_name="s")

  @functools.partial(
      pl.kernel,
      out_type=jax.ShapeDtypeStruct((_B, _D), jnp.float32),
      mesh=mesh,
      scratch_types=[
          pltpu.VMEM((_JC, _NCHUNK, _CHUNK), jnp.int32),
          pltpu.VMEM((_BPW, _D), jnp.float32),
          pltpu.SemaphoreType.DMA,
      ],
  )
  def pool(text_hbm, emb_hbm, out_hbm, idx_v, acc_v, sem):
    wid = lax.axis_index("s") * _NC + lax.axis_index("c")
    gbase = wid * _NCHUNK

    for h in range(_NJ):
      pltpu.sync_copy(
          text_hbm.at[pl.ds(h * _JC, _JC), pl.ds(gbase, _NCHUNK)], idx_v)

      if h == 0:
        for c in range(_NCHUNK):
          pltpu.async_copy(emb_hbm.at[idx_v.at[0, c]],
                           acc_v.at[pl.ds(c * _CHUNK, _CHUNK)], sem)
        for c in range(_NCHUNK):
          pltpu.make_async_copy(emb_hbm.at[idx_v.at[0, c]],
                                acc_v.at[pl.ds(c * _CHUNK, _CHUNK)],
                                sem).wait()

      @pl.loop(1 if h == 0 else 0, _JC)
      def _(j):
        for c in range(_NCHUNK):
          pltpu.async_copy(emb_hbm.at[idx_v.at[j, c]],
                           acc_v.at[pl.ds(c * _CHUNK, _CHUNK)], sem, add=True)
        for c in range(_NCHUNK):
          pltpu.make_async_copy(emb_hbm.at[idx_v.at[j, c]],
                                acc_v.at[pl.ds(c * _CHUNK, _CHUNK)],
                                sem).wait()

    pltpu.sync_copy(acc_v, out_hbm.at[pl.ds(wid * _BPW, _BPW)])

  return pool(text_r, emb)


def _mlp_block(acc_ref, text_ref, emb0_ref, w1_ref, b1_ref, w2_ref, b2_ref,
               wo_ref, bo_ref, out_ref):
  cnt = jnp.sum((text_ref[...] == 0).astype(jnp.float32), axis=1,
                keepdims=True)
  pooled = (acc_ref[...] - cnt * emb0_ref[...]) * (1.0 / _SEQ)
  h = jnp.dot(pooled, w1_ref[...], preferred_element_type=jnp.float32)
  h = jnp.maximum(h + b1_ref[...], 0.0)
  h = jnp.dot(h, w2_ref[...], preferred_element_type=jnp.float32)
  h = jnp.maximum(h + b2_ref[...], 0.0)
  out_ref[...] = (
      jnp.dot(h, wo_ref[...], preferred_element_type=jnp.float32)
      + bo_ref[...])


def _tc_mlp(acc, text, emb0, W1, b1, W2, b2, Wo, bo):
  bblk = 2048
  grid = (_B // bblk,)
  full = lambda shape: pl.BlockSpec(shape, lambda i: (0, 0))
  return pl.pallas_call(
      _mlp_block,
      grid=grid,
      in_specs=[
          pl.BlockSpec((bblk, _D), lambda i: (i, 0)),
          pl.BlockSpec((bblk, _SEQ), lambda i: (i, 0)),
          full((1, _D)),
          full(W1.shape),
          full((1, 256)),
          full(W2.shape),
          full((1, 128)),
          full(Wo.shape),
          full((1, 1)),
      ],
      out_specs=pl.BlockSpec((bblk, 1), lambda i: (i, 0)),
      out_shape=jax.ShapeDtypeStruct((_B, 1), jnp.float32),
  )(acc, text, emb0, W1, b1.reshape(1, -1), W2, b2.reshape(1, -1), Wo,
    bo.reshape(1, -1))


def kernel(text, emb, W1, b1, W2, b2, Wo, bo):
  text_r = text.T.reshape(_SEQ, _B // _CHUNK, _CHUNK)
  acc = _sc_pool(text_r, emb)
  return _tc_mlp(acc, text, emb[0:1], W1, b1, W2, b2, Wo, bo)


# speedup vs baseline: 3.0603x; 3.0603x over previous
import functools

import jax
import jax.numpy as jnp
from jax import lax
from jax.experimental import pallas as pl
from jax.experimental.pallas import tpu as pltpu
from jax.experimental.pallas import tpu_sc as plsc

_VOCAB = 1000000
_D = 64
_B = 16384
_SEQ = 200

_NC = 2
_NS = 16
_NW = _NC * _NS
_BPW = _B // _NW
_CHUNK = 128
_NCHUNK = _BPW // _CHUNK
_JC = 50
_NJ = _SEQ // _JC


def _sc_pool(text_r, emb):
  mesh = plsc.VectorSubcoreMesh(core_axis_name="c", subcore_axis_name="s")

  @functools.partial(
      pl.kernel,
      out_type=jax.ShapeDtypeStruct((_B, _D), jnp.float32),
      mesh=mesh,
      scratch_types=[
          pltpu.VMEM((_JC, _NCHUNK, _CHUNK), jnp.int32),
          pltpu.VMEM((_BPW, _D), jnp.float32),
          pltpu.SemaphoreType.DMA,
      ],
      compiler_params=pltpu.CompilerParams(use_tc_tiling_on_sc=False),
  )
  def pool(text_hbm, emb_hbm, out_hbm, idx_v, acc_v, sem):
    wid = lax.axis_index("s") * _NC + lax.axis_index("c")
    gbase = wid * _NCHUNK

    for h in range(_NJ):
      pltpu.sync_copy(
          text_hbm.at[pl.ds(h * _JC, _JC), pl.ds(gbase, _NCHUNK)], idx_v)

      if h == 0:
        for c in range(_NCHUNK):
          pltpu.async_copy(emb_hbm.at[idx_v.at[0, c]],
                           acc_v.at[pl.ds(c * _CHUNK, _CHUNK)], sem)
        for c in range(_NCHUNK):
          pltpu.make_async_copy(emb_hbm.at[idx_v.at[0, c]],
                                acc_v.at[pl.ds(c * _CHUNK, _CHUNK)],
                                sem).wait()

      @pl.loop(1 if h == 0 else 0, _JC)
      def _(j):
        for c in range(_NCHUNK):
          pltpu.async_copy(emb_hbm.at[idx_v.at[j, c]],
                           acc_v.at[pl.ds(c * _CHUNK, _CHUNK)], sem, add=True)
        for c in range(_NCHUNK):
          pltpu.make_async_copy(emb_hbm.at[idx_v.at[j, c]],
                                acc_v.at[pl.ds(c * _CHUNK, _CHUNK)],
                                sem).wait()

    pltpu.sync_copy(acc_v, out_hbm.at[pl.ds(wid * _BPW, _BPW)])

  return pool(text_r, emb)


def _mlp_block(acc_ref, text_ref, emb0_ref, w1_ref, b1_ref, w2_ref, b2_ref,
               wo_ref, bo_ref, out_ref):
  cnt = jnp.sum((text_ref[...] == 0).astype(jnp.float32), axis=1,
                keepdims=True)
  pooled = (acc_ref[...] - cnt * emb0_ref[...]) * (1.0 / _SEQ)
  h = jnp.dot(pooled, w1_ref[...], preferred_element_type=jnp.float32)
  h = jnp.maximum(h + b1_ref[...], 0.0)
  h = jnp.dot(h, w2_ref[...], preferred_element_type=jnp.float32)
  h = jnp.maximum(h + b2_ref[...], 0.0)
  out_ref[...] = (
      jnp.dot(h, wo_ref[...], preferred_element_type=jnp.float32)
      + bo_ref[...])


def _tc_mlp(acc, text, emb0, W1, b1, W2, b2, Wo, bo):
  bblk = 2048
  grid = (_B // bblk,)
  full = lambda shape: pl.BlockSpec(shape, lambda i: (0, 0))
  return pl.pallas_call(
      _mlp_block,
      grid=grid,
      in_specs=[
          pl.BlockSpec((bblk, _D), lambda i: (i, 0)),
          pl.BlockSpec((bblk, _SEQ), lambda i: (i, 0)),
          full((1, _D)),
          full(W1.shape),
          full((1, 256)),
          full(W2.shape),
          full((1, 128)),
          full(Wo.shape),
          full((1, 1)),
      ],
      out_specs=pl.BlockSpec((bblk, 1), lambda i: (i, 0)),
      out_shape=jax.ShapeDtypeStruct((_B, 1), jnp.float32),
  )(acc, text, emb0, W1, b1.reshape(1, -1), W2, b2.reshape(1, -1), Wo,
    bo.reshape(1, -1))


def kernel(text, emb, W1, b1, W2, b2, Wo, bo):
  text_r = text.T.reshape(_SEQ, _B // _CHUNK, _CHUNK)
  acc = _sc_pool(text_r, emb)
  return _tc_mlp(acc, text, emb[0:1], W1, b1, W2, b2, Wo, bo)
